# Initial kernel scaffold; baseline (speedup 1.0000x reference)
import jax
import jax.numpy as jnp
from jax.experimental import pallas as pl


def kernel(x, edge_index, W1, b1, W2, b2):
    raise NotImplementedError("write your pallas kernel here")


# speedup vs baseline: 11.9388x; 11.9388x over previous
import functools

import jax
import jax.numpy as jnp
from jax import lax
from jax.experimental import pallas as pl
from jax.experimental.pallas import tpu as pltpu
from jax.experimental.pallas import tpu_sc as plsc

N = 10000
D = 128
E = 320000

N_PAD = 10240
NW = 32
CHUNK = 128
CPT = -(-E // (NW * CHUNK))
E_PAD = NW * CPT * CHUNK
ROWS_PER_TILE = N_PAD // 16
DEG_W = 128
TC_BLOCK = 512
N_TC_BLOCKS = N_PAD // TC_BLOCK

_MESH = plsc.VectorSubcoreMesh(
    core_axis_name="c", subcore_axis_name="s", num_cores=2, num_subcores=16)


def _sc_degree_body(dst_hbm, zeros_hbm, ones_hbm, out_hbm, deg_sp, idx_v, ones_v):
    c = lax.axis_index("c")
    s = lax.axis_index("s")
    wid = s * 2 + c
    r0 = s * ROWS_PER_TILE
    pltpu.sync_copy(zeros_hbm.at[pl.ds(r0, ROWS_PER_TILE)],
                    deg_sp.at[pl.ds(r0, ROWS_PER_TILE)])
    pltpu.sync_copy(ones_hbm, ones_v)
    pltpu.sync_copy(dst_hbm.at[wid], idx_v)
    plsc.subcore_barrier()

    def body(j, carry):
        pltpu.sync_copy(ones_v, deg_sp.at[idx_v.at[j]], add=True)
        return carry

    lax.fori_loop(0, CPT, body, 0)
    plsc.subcore_barrier()
    pltpu.sync_copy(deg_sp.at[pl.ds(r0, ROWS_PER_TILE)],
                    out_hbm.at[c, pl.ds(r0, ROWS_PER_TILE)])


_sc_degree = pl.kernel(
    _sc_degree_body,
    out_type=jax.ShapeDtypeStruct((2, N_PAD, DEG_W), jnp.float32),
    mesh=_MESH,
    scratch_types=[
        pltpu.VMEM_SHARED((N_PAD, DEG_W), jnp.float32),
        pltpu.VMEM((CPT, CHUNK), jnp.int32),
        pltpu.VMEM((CHUNK, DEG_W), jnp.float32),
    ],
)


def _sc_prop_body(hs_hbm, src_hbm, dst_hbm, zeros_hbm, out_hbm,
                  agg_sp, src_v, dst_v, rows_v, sem):
    c = lax.axis_index("c")
    s = lax.axis_index("s")
    wid = s * 2 + c
    r0 = s * ROWS_PER_TILE
    pltpu.sync_copy(zeros_hbm.at[pl.ds(r0, ROWS_PER_TILE)],
                    agg_sp.at[pl.ds(r0, ROWS_PER_TILE)])
    pltpu.sync_copy(src_hbm.at[wid], src_v)
    pltpu.sync_copy(dst_hbm.at[wid], dst_v)
    plsc.subcore_barrier()

    def body(j, carry):
        pltpu.async_copy(hs_hbm.at[src_v.at[j]], rows_v, sem).wait()
        pltpu.sync_copy(rows_v, agg_sp.at[dst_v.at[j]], add=True)
        return carry

    lax.fori_loop(0, CPT, body, 0)
    plsc.subcore_barrier()
    pltpu.sync_copy(agg_sp.at[pl.ds(r0, ROWS_PER_TILE)],
                    out_hbm.at[c, pl.ds(r0, ROWS_PER_TILE)])


_sc_prop = pl.kernel(
    _sc_prop_body,
    out_type=jax.ShapeDtypeStruct((2, N_PAD, D), jnp.float32),
    mesh=_MESH,
    scratch_types=[
        pltpu.VMEM_SHARED((N_PAD, D), jnp.float32),
        pltpu.VMEM((CPT, CHUNK), jnp.int32),
        pltpu.VMEM((CPT, CHUNK), jnp.int32),
        pltpu.VMEM((CHUNK, D), jnp.float32),
        pltpu.SemaphoreType.DMA,
    ],
)


def _dis_from(deg_ref):
    d = deg_ref[0, 0, :, 0:1] + deg_ref[1, 0, :, 0:1] + 1.0
    return lax.rsqrt(d)


def _tc_mm1_body(x_ref, w_ref, deg_ref, out_ref):
    dis = _dis_from(deg_ref)
    h = jnp.dot(x_ref[...], w_ref[...], preferred_element_type=jnp.float32)
    out_ref[...] = h * dis


def _tc_mm2_body(p_ref, hs_ref, deg_ref, b_ref, w_ref, out_ref):
    dis = _dis_from(deg_ref)
    t = dis * (p_ref[0] + p_ref[1] + hs_ref[...]) + b_ref[...]
    t = jnp.maximum(t, 0.0)
    h = jnp.dot(t, w_ref[...], preferred_element_type=jnp.float32)
    out_ref[...] = h * dis


def _tc_comb_body(p_ref, hs_ref, deg_ref, b_ref, out_ref):
    dis = _dis_from(deg_ref)
    out_ref[...] = dis * (p_ref[0] + p_ref[1] + hs_ref[...]) + b_ref[...]


_row_spec = pl.BlockSpec((TC_BLOCK, D), lambda i: (i, 0))
_w_spec = pl.BlockSpec((D, D), lambda i: (0, 0))
_deg_spec = pl.BlockSpec((2, 1, TC_BLOCK, DEG_W), lambda i: (0, i, 0, 0))
_p_spec = pl.BlockSpec((2, TC_BLOCK, D), lambda i: (0, i, 0))
_b_spec = pl.BlockSpec((1, D), lambda i: (0, 0))
_row_out = jax.ShapeDtypeStruct((N_PAD, D), jnp.float32)

_tc_mm1 = pl.pallas_call(
    _tc_mm1_body,
    grid=(N_TC_BLOCKS,),
    in_specs=[_row_spec, _w_spec, _deg_spec],
    out_specs=_row_spec,
    out_shape=_row_out,
)

_tc_mm2 = pl.pallas_call(
    _tc_mm2_body,
    grid=(N_TC_BLOCKS,),
    in_specs=[_p_spec, _row_spec, _deg_spec, _b_spec, _w_spec],
    out_specs=_row_spec,
    out_shape=_row_out,
)

_tc_comb = pl.pallas_call(
    _tc_comb_body,
    grid=(N_TC_BLOCKS,),
    in_specs=[_p_spec, _row_spec, _deg_spec, _b_spec],
    out_specs=_row_spec,
    out_shape=_row_out,
)


@jax.jit
def kernel(x, edge_index, W1, b1, W2, b2):
    src = edge_index[0]
    dst = edge_index[1]
    pad = E_PAD - E
    src_p = jnp.concatenate(
        [src, jnp.zeros((pad,), jnp.int32)]).reshape(NW, CPT, CHUNK)
    dst_p = jnp.concatenate(
        [dst, jnp.full((pad,), N, jnp.int32)]).reshape(NW, CPT, CHUNK)
    x_p = jnp.pad(x, ((0, N_PAD - N), (0, 0)))
    zeros_deg = jnp.zeros((N_PAD, DEG_W), jnp.float32)
    ones_pay = jnp.ones((CHUNK, DEG_W), jnp.float32)
    zeros_tbl = jnp.zeros((N_PAD, D), jnp.float32)

    deg = _sc_degree(dst_p, zeros_deg, ones_pay)
    deg4 = deg.reshape(2, N_TC_BLOCKS, TC_BLOCK, DEG_W)

    hs1 = _tc_mm1(x_p, W1, deg4)
    p1 = _sc_prop(hs1, src_p, dst_p, zeros_tbl)
    hs2 = _tc_mm2(p1, hs1, deg4, b1.reshape(1, D), W2)
    p2 = _sc_prop(hs2, src_p, dst_p, zeros_tbl)
    out = _tc_comb(p2, hs2, deg4, b2.reshape(1, D))
    return out[:N]


# speedup vs baseline: 19.2725x; 1.6143x over previous
import functools

import jax
import jax.numpy as jnp
from jax import lax
from jax.experimental import pallas as pl
from jax.experimental.pallas import tpu as pltpu
from jax.experimental.pallas import tpu_sc as plsc

N = 10000
D = 128
E = 320000

N_PAD = 10240
NW = 32
CHUNK = 128
CPT = -(-E // (NW * CHUNK))
E_PAD = NW * CPT * CHUNK
ROWS_PER_TILE = N_PAD // 16
DEG_W = 128
TC_BLOCK = 512
N_TC_BLOCKS = N_PAD // TC_BLOCK

_MESH = plsc.VectorSubcoreMesh(
    core_axis_name="c", subcore_axis_name="s", num_cores=2, num_subcores=16)


def _sc_degree_body(dst_hbm, zeros_hbm, ones_hbm, out_hbm, deg_sp, idx_v, ones_v):
    c = lax.axis_index("c")
    s = lax.axis_index("s")
    wid = s * 2 + c
    r0 = s * ROWS_PER_TILE
    pltpu.sync_copy(zeros_hbm.at[pl.ds(r0, ROWS_PER_TILE)],
                    deg_sp.at[pl.ds(r0, ROWS_PER_TILE)])
    pltpu.sync_copy(ones_hbm, ones_v)
    pltpu.sync_copy(dst_hbm.at[wid], idx_v)
    plsc.subcore_barrier()

    def body(j, carry):
        pltpu.sync_copy(ones_v, deg_sp.at[idx_v.at[j]], add=True)
        return carry

    lax.fori_loop(0, CPT, body, 0)
    plsc.subcore_barrier()
    pltpu.sync_copy(deg_sp.at[pl.ds(r0, ROWS_PER_TILE)],
                    out_hbm.at[c, pl.ds(r0, ROWS_PER_TILE)])


_sc_degree = pl.kernel(
    _sc_degree_body,
    out_type=jax.ShapeDtypeStruct((2, N_PAD, DEG_W), jnp.float32),
    mesh=_MESH,
    scratch_types=[
        pltpu.VMEM_SHARED((N_PAD, DEG_W), jnp.float32),
        pltpu.VMEM((CPT, CHUNK), jnp.int32),
        pltpu.VMEM((CHUNK, DEG_W), jnp.float32),
    ],
)


def _sc_prop_body(hs_hbm, src_hbm, dst_hbm, zeros_hbm, out_hbm,
                  agg_sp, src_v, dst_v, rows_v, sem):
    c = lax.axis_index("c")
    s = lax.axis_index("s")
    wid = s * 2 + c
    r0 = s * ROWS_PER_TILE
    pltpu.sync_copy(zeros_hbm.at[pl.ds(r0, ROWS_PER_TILE)],
                    agg_sp.at[pl.ds(r0, ROWS_PER_TILE)])
    pltpu.sync_copy(src_hbm.at[wid], src_v)
    pltpu.sync_copy(dst_hbm.at[wid], dst_v)
    plsc.subcore_barrier()

    def body(j, carry):
        pltpu.async_copy(hs_hbm.at[src_v.at[j]], rows_v, sem).wait()
        pltpu.sync_copy(rows_v, agg_sp.at[dst_v.at[j]], add=True)
        return carry

    lax.fori_loop(0, CPT, body, 0)
    plsc.subcore_barrier()
    pltpu.sync_copy(agg_sp.at[pl.ds(r0, ROWS_PER_TILE)],
                    out_hbm.at[c, pl.ds(r0, ROWS_PER_TILE)])


_sc_prop = pl.kernel(
    _sc_prop_body,
    out_type=jax.ShapeDtypeStruct((2, N_PAD, D), jnp.float32),
    mesh=_MESH,
    scratch_types=[
        pltpu.VMEM_SHARED((N_PAD, D), jnp.float32),
        pltpu.VMEM((CPT, CHUNK), jnp.int32),
        pltpu.VMEM((CPT, CHUNK), jnp.int32),
        pltpu.VMEM((CHUNK, D), jnp.float32),
        pltpu.SemaphoreType.DMA,
    ],
)


def _dis_from(deg_ref):
    d = deg_ref[0, 0, :, 0:1] + deg_ref[1, 0, :, 0:1] + 1.0
    return lax.rsqrt(d)


def _tc_mm1_body(x_ref, w_ref, deg_ref, out_ref):
    dis = _dis_from(deg_ref)
    h = jnp.dot(x_ref[...], w_ref[...], preferred_element_type=jnp.float32)
    out_ref[...] = h * dis


def _tc_mm2_body(p_ref, hs_ref, deg_ref, b_ref, w_ref, out_ref):
    dis = _dis_from(deg_ref)
    t = dis * (p_ref[0] + p_ref[1] + hs_ref[...]) + b_ref[...]
    t = jnp.maximum(t, 0.0)
    h = jnp.dot(t, w_ref[...], preferred_element_type=jnp.float32)
    out_ref[...] = h * dis


def _tc_comb_body(p_ref, hs_ref, deg_ref, b_ref, out_ref):
    dis = _dis_from(deg_ref)
    out_ref[...] = dis * (p_ref[0] + p_ref[1] + hs_ref[...]) + b_ref[...]


_row_spec = pl.BlockSpec((TC_BLOCK, D), lambda i: (i, 0))
_w_spec = pl.BlockSpec((D, D), lambda i: (0, 0))
_deg_spec = pl.BlockSpec((2, 1, TC_BLOCK, DEG_W), lambda i: (0, i, 0, 0))
_p_spec = pl.BlockSpec((2, TC_BLOCK, D), lambda i: (0, i, 0))
_b_spec = pl.BlockSpec((1, D), lambda i: (0, 0))
_row_out = jax.ShapeDtypeStruct((N_PAD, D), jnp.float32)

_tc_mm1 = pl.pallas_call(
    _tc_mm1_body,
    grid=(N_TC_BLOCKS,),
    in_specs=[_row_spec, _w_spec, _deg_spec],
    out_specs=_row_spec,
    out_shape=_row_out,
)

_tc_mm2 = pl.pallas_call(
    _tc_mm2_body,
    grid=(N_TC_BLOCKS,),
    in_specs=[_p_spec, _row_spec, _deg_spec, _b_spec, _w_spec],
    out_specs=_row_spec,
    out_shape=_row_out,
)

_tc_comb = pl.pallas_call(
    _tc_comb_body,
    grid=(N_TC_BLOCKS,),
    in_specs=[_p_spec, _row_spec, _deg_spec, _b_spec],
    out_specs=_row_spec,
    out_shape=_row_out,
)


@jax.jit
def kernel(x, edge_index, W1, b1, W2, b2):
    src = edge_index[0]
    dst = edge_index[1]
    pad = E_PAD - E
    pad_iota = jnp.arange(pad, dtype=jnp.int32)
    src_p = jnp.concatenate(
        [src, (pad_iota * 37) % N]).reshape(NW, CPT, CHUNK)
    dst_p = jnp.concatenate(
        [dst, N + pad_iota % (N_PAD - N)]).reshape(NW, CPT, CHUNK)
    x_p = jnp.pad(x, ((0, N_PAD - N), (0, 0)))
    zeros_deg = jnp.zeros((N_PAD, DEG_W), jnp.float32)
    ones_pay = jnp.ones((CHUNK, DEG_W), jnp.float32)
    zeros_tbl = jnp.zeros((N_PAD, D), jnp.float32)

    deg = _sc_degree(dst_p, zeros_deg, ones_pay)
    deg4 = deg.reshape(2, N_TC_BLOCKS, TC_BLOCK, DEG_W)

    hs1 = _tc_mm1(x_p, W1, deg4)
    p1 = _sc_prop(hs1, src_p, dst_p, zeros_tbl)
    hs2 = _tc_mm2(p1, hs1, deg4, b1.reshape(1, D), W2)
    p2 = _sc_prop(hs2, src_p, dst_p, zeros_tbl)
    out = _tc_comb(p2, hs2, deg4, b2.reshape(1, D))
    return out[:N]


# speedup vs baseline: 25.6944x; 1.3332x over previous
import functools

import jax
import jax.numpy as jnp
from jax import lax
from jax.experimental import pallas as pl
from jax.experimental.pallas import tpu as pltpu
from jax.experimental.pallas import tpu_sc as plsc

N = 10000
D = 128
E = 320000

N_PAD = 10240
NW = 32
CHUNK = 128
CPT = 80
PHASE = CPT // 2
E_PAD = NW * CPT * CHUNK
ROWS_PER_TILE = N_PAD // 16
DEG_W = 128
TC_BLOCK = 512
N_TC_BLOCKS = N_PAD // TC_BLOCK

_MESH = plsc.VectorSubcoreMesh(
    core_axis_name="c", subcore_axis_name="s", num_cores=2, num_subcores=16)


def _sc_degree_body(dst_hbm, zeros_hbm, ones_hbm, out_hbm, deg_sp, idx_v, ones_v,
                    sem):
    c = lax.axis_index("c")
    s = lax.axis_index("s")
    wid = s * 2 + c
    r0 = s * ROWS_PER_TILE
    pltpu.sync_copy(zeros_hbm.at[pl.ds(r0, ROWS_PER_TILE)],
                    deg_sp.at[pl.ds(r0, ROWS_PER_TILE)])
    pltpu.sync_copy(ones_hbm, ones_v)
    pltpu.sync_copy(dst_hbm.at[wid], idx_v)
    plsc.subcore_barrier()

    def body(j, carry):
        pltpu.async_copy(ones_v, deg_sp.at[idx_v.at[j]], sem, add=True)
        return carry

    lax.fori_loop(0, CPT, body, 0)

    def drain(j, carry):
        pltpu.make_async_copy(ones_v, deg_sp.at[idx_v.at[j]], sem).wait()
        return carry

    lax.fori_loop(0, CPT, drain, 0)
    plsc.subcore_barrier()
    pltpu.sync_copy(deg_sp.at[pl.ds(r0, ROWS_PER_TILE)],
                    out_hbm.at[c, pl.ds(r0, ROWS_PER_TILE)])


_sc_degree = pl.kernel(
    _sc_degree_body,
    out_type=jax.ShapeDtypeStruct((2, N_PAD, DEG_W), jnp.float32),
    mesh=_MESH,
    scratch_types=[
        pltpu.VMEM_SHARED((N_PAD, DEG_W), jnp.float32),
        pltpu.VMEM((CPT, CHUNK), jnp.int32),
        pltpu.VMEM((CHUNK, DEG_W), jnp.float32),
        pltpu.SemaphoreType.DMA,
    ],
)


def _sc_prop_body(hs_hbm, src_hbm, dst_hbm, zeros_hbm, out_hbm,
                  agg_sp, src_v, dst_v, rows_v, sem_g, sem_s):
    c = lax.axis_index("c")
    s = lax.axis_index("s")
    wid = s * 2 + c
    r0 = s * ROWS_PER_TILE
    pltpu.sync_copy(zeros_hbm.at[pl.ds(r0, ROWS_PER_TILE)],
                    agg_sp.at[pl.ds(r0, ROWS_PER_TILE)])
    plsc.subcore_barrier()

    for p in range(2):
        pltpu.sync_copy(src_hbm.at[wid, pl.ds(p * PHASE, PHASE)], src_v)
        pltpu.sync_copy(dst_hbm.at[wid, pl.ds(p * PHASE, PHASE)], dst_v)
        pltpu.async_copy(hs_hbm.at[src_v.at[0]], rows_v.at[0], sem_g.at[0])

        def body(g, carry):
            b = lax.rem(g, 2)
            nb = 1 - b

            @pl.when(g >= 1)
            def _():
                pltpu.make_async_copy(rows_v.at[nb],
                                      agg_sp.at[dst_v.at[g - 1]],
                                      sem_s.at[nb]).wait()

            @pl.when(g + 1 < PHASE)
            def _():
                pltpu.async_copy(hs_hbm.at[src_v.at[g + 1]], rows_v.at[nb],
                                 sem_g.at[nb])

            pltpu.make_async_copy(hs_hbm.at[src_v.at[g]], rows_v.at[b],
                                  sem_g.at[b]).wait()
            pltpu.async_copy(rows_v.at[b], agg_sp.at[dst_v.at[g]],
                             sem_s.at[b], add=True)
            return carry

        lax.fori_loop(0, PHASE, body, 0)
        last = PHASE - 1
        pltpu.make_async_copy(rows_v.at[last % 2],
                              agg_sp.at[dst_v.at[last]],
                              sem_s.at[last % 2]).wait()
    plsc.subcore_barrier()
    pltpu.sync_copy(agg_sp.at[pl.ds(r0, ROWS_PER_TILE)],
                    out_hbm.at[c, pl.ds(r0, ROWS_PER_TILE)])


_sc_prop = pl.kernel(
    _sc_prop_body,
    out_type=jax.ShapeDtypeStruct((2, N_PAD, D), jnp.float32),
    mesh=_MESH,
    scratch_types=[
        pltpu.VMEM_SHARED((N_PAD, D), jnp.float32),
        pltpu.VMEM((PHASE, CHUNK), jnp.int32),
        pltpu.VMEM((PHASE, CHUNK), jnp.int32),
        pltpu.VMEM((2, CHUNK, D), jnp.float32),
        pltpu.SemaphoreType.DMA((2,)),
        pltpu.SemaphoreType.DMA((2,)),
    ],
)


def _dis_from(deg_ref):
    d = deg_ref[0, 0, :, 0:1] + deg_ref[1, 0, :, 0:1] + 1.0
    return lax.rsqrt(d)


def _tc_mm1_body(x_ref, w_ref, deg_ref, out_ref):
    dis = _dis_from(deg_ref)
    h = jnp.dot(x_ref[...], w_ref[...], preferred_element_type=jnp.float32)
    out_ref[...] = h * dis


def _tc_mm2_body(p_ref, hs_ref, deg_ref, b_ref, w_ref, out_ref):
    dis = _dis_from(deg_ref)
    t = dis * (p_ref[0] + p_ref[1] + hs_ref[...]) + b_ref[...]
    t = jnp.maximum(t, 0.0)
    h = jnp.dot(t, w_ref[...], preferred_element_type=jnp.float32)
    out_ref[...] = h * dis


def _tc_comb_body(p_ref, hs_ref, deg_ref, b_ref, out_ref):
    dis = _dis_from(deg_ref)
    out_ref[...] = dis * (p_ref[0] + p_ref[1] + hs_ref[...]) + b_ref[...]


_row_spec = pl.BlockSpec((TC_BLOCK, D), lambda i: (i, 0))
_w_spec = pl.BlockSpec((D, D), lambda i: (0, 0))
_deg_spec = pl.BlockSpec((2, 1, TC_BLOCK, DEG_W), lambda i: (0, i, 0, 0))
_p_spec = pl.BlockSpec((2, TC_BLOCK, D), lambda i: (0, i, 0))
_b_spec = pl.BlockSpec((1, D), lambda i: (0, 0))
_row_out = jax.ShapeDtypeStruct((N_PAD, D), jnp.float32)

_tc_mm1 = pl.pallas_call(
    _tc_mm1_body,
    grid=(N_TC_BLOCKS,),
    in_specs=[_row_spec, _w_spec, _deg_spec],
    out_specs=_row_spec,
    out_shape=_row_out,
)

_tc_mm2 = pl.pallas_call(
    _tc_mm2_body,
    grid=(N_TC_BLOCKS,),
    in_specs=[_p_spec, _row_spec, _deg_spec, _b_spec, _w_spec],
    out_specs=_row_spec,
    out_shape=_row_out,
)

_tc_comb = pl.pallas_call(
    _tc_comb_body,
    grid=(N_TC_BLOCKS,),
    in_specs=[_p_spec, _row_spec, _deg_spec, _b_spec],
    out_specs=_row_spec,
    out_shape=_row_out,
)


@jax.jit
def kernel(x, edge_index, W1, b1, W2, b2):
    src = edge_index[0]
    dst = edge_index[1]
    pad = E_PAD - E
    pad_iota = jnp.arange(pad, dtype=jnp.int32)
    src_p = jnp.concatenate(
        [src, (pad_iota * 37) % N]).reshape(NW, CPT, CHUNK)
    dst_p = jnp.concatenate(
        [dst, N + pad_iota % (N_PAD - N)]).reshape(NW, CPT, CHUNK)
    x_p = jnp.pad(x, ((0, N_PAD - N), (0, 0)))
    zeros_deg = jnp.zeros((N_PAD, DEG_W), jnp.float32)
    ones_pay = jnp.ones((CHUNK, DEG_W), jnp.float32)
    zeros_tbl = jnp.zeros((N_PAD, D), jnp.float32)

    deg = _sc_degree(dst_p, zeros_deg, ones_pay)
    deg4 = deg.reshape(2, N_TC_BLOCKS, TC_BLOCK, DEG_W)

    hs1 = _tc_mm1(x_p, W1, deg4)
    p1 = _sc_prop(hs1, src_p, dst_p, zeros_tbl)
    hs2 = _tc_mm2(p1, hs1, deg4, b1.reshape(1, D), W2)
    p2 = _sc_prop(hs2, src_p, dst_p, zeros_tbl)
    out = _tc_comb(p2, hs2, deg4, b2.reshape(1, D))
    return out[:N]
